# Initial kernel scaffold; baseline (speedup 1.0000x reference)
import jax
import jax.numpy as jnp
from jax.experimental import pallas as pl


def kernel(user_emb, item_emb, adj_indices, adj_values):
    raise NotImplementedError("write your pallas kernel here")


# speedup vs baseline: 10.7486x; 10.7486x over previous
import functools

import jax
import jax.numpy as jnp
from jax import lax
from jax.experimental import pallas as pl
from jax.experimental.pallas import tpu as pltpu
from jax.experimental.pallas import tpu_sc as plsc

NUM_USERS = 25000
NUM_ITEMS = 25000
N = NUM_USERS + NUM_ITEMS
D = 32
E = 1600000
NUM_LAYERS = 3
DROPOUT_P = 0.2

NC, NS, L = 2, 16, 16
N_PAD = 50176
NW = NC * NS
CH = 128
GROUP = 8
R_PER_W = 392
NGROUPS = R_PER_W // GROUP
E_PAD = NW * R_PER_W * CH
ROWS_PER_TILE = N_PAD // NS
ZCH = 392
NZ = ROWS_PER_TILE // ZCH

_mesh = plsc.VectorSubcoreMesh(core_axis_name="c", subcore_axis_name="s")


@functools.partial(
    pl.kernel,
    out_type=(
        jax.ShapeDtypeStruct((N_PAD, D), jnp.float32),
        jax.ShapeDtypeStruct((N_PAD, D), jnp.float32),
    ),
    mesh=_mesh,
    compiler_params=pltpu.CompilerParams(use_tc_tiling_on_sc=False),
    scratch_types=dict(
        acc=pltpu.VMEM_SHARED((N_PAD, D), jnp.float32),
        src_v=pltpu.VMEM((GROUP, CH), jnp.int32),
        dst_v=pltpu.VMEM((GROUP, CH), jnp.int32),
        val_v=pltpu.VMEM((GROUP, CH), jnp.float32),
        rows_v=pltpu.VMEM((CH, D), jnp.float32),
        stage_v=pltpu.VMEM((ZCH, D), jnp.float32),
        sem=pltpu.SemaphoreType.DMA,
    ),
)
def _propagate(x_hbm, src_hbm, dst_hbm, val_hbm, p0_hbm, p1_hbm,
               acc, src_v, dst_v, val_v, rows_v, stage_v, sem):
    c = lax.axis_index("c")
    s = lax.axis_index("s")
    w = c * NS + s
    base_out = s * ROWS_PER_TILE

    def _zrow(i, carry):
        stage_v[i, pl.ds(0, L)] = jnp.zeros((L,), jnp.float32)
        stage_v[i, pl.ds(L, L)] = jnp.zeros((L,), jnp.float32)
        return carry

    lax.fori_loop(0, ZCH, _zrow, 0)

    def _zchunk(k, carry):
        pltpu.sync_copy(stage_v, acc.at[pl.ds(base_out + k * ZCH, ZCH)])
        return carry

    lax.fori_loop(0, NZ, _zchunk, 0)
    plsc.subcore_barrier()

    row0 = w * R_PER_W

    def _group(g, carry):
        gr = row0 + g * GROUP
        pltpu.sync_copy(src_hbm.at[pl.ds(gr, GROUP)], src_v)
        pltpu.sync_copy(dst_hbm.at[pl.ds(gr, GROUP)], dst_v)
        pltpu.sync_copy(val_hbm.at[pl.ds(gr, GROUP)], val_v)

        def _sub(j, carry2):
            pltpu.async_copy(x_hbm.at[src_v.at[j]], rows_v, sem).wait()

            def _scale_blk(b, carry3):
                vals16 = val_v[j, pl.ds(b * L, L)]
                for l in range(L):
                    i = b * L + l
                    v = vals16[l]
                    rows_v[i, pl.ds(0, L)] = rows_v[i, pl.ds(0, L)] * v
                    rows_v[i, pl.ds(L, L)] = rows_v[i, pl.ds(L, L)] * v
                return carry3

            lax.fori_loop(0, CH // L, _scale_blk, 0)
            pltpu.sync_copy(rows_v, acc.at[dst_v.at[j]], add=True)
            return carry2

        lax.fori_loop(0, GROUP, _sub, 0)
        return carry

    lax.fori_loop(0, NGROUPS, _group, 0)
    plsc.subcore_barrier()

    def _dump(out_hbm):
        def _dchunk(k, carry):
            off = base_out + k * ZCH
            pltpu.sync_copy(acc.at[pl.ds(off, ZCH)], stage_v)
            pltpu.sync_copy(stage_v, out_hbm.at[pl.ds(off, ZCH)])
            return carry

        lax.fori_loop(0, NZ, _dchunk, 0)

    @pl.when(c == 0)
    def _():
        _dump(p0_hbm)

    @pl.when(c == 1)
    def _():
        _dump(p1_hbm)


def kernel(user_emb, item_emb, adj_indices, adj_values):
    all_emb = jnp.concatenate([user_emb, item_emb], axis=0)
    dkey = jax.random.key(12345)
    keep = jax.random.bernoulli(dkey, 1.0 - DROPOUT_P, all_emb.shape)
    x0 = jnp.where(keep, all_emb / (1.0 - DROPOUT_P), 0.0)

    idx = adj_indices.astype(jnp.int32)
    pad = E_PAD - E
    src_p = jnp.pad(idx[1], (0, pad)).reshape(-1, CH)
    dst_p = jnp.pad(idx[0], (0, pad)).reshape(-1, CH)
    val_p = jnp.pad(adj_values, (0, pad)).reshape(-1, CH)

    x = x0
    total = x0
    for _ in range(NUM_LAYERS):
        p0, p1 = _propagate(x, src_p, dst_p, val_p)
        x = (p0 + p1)[:N]
        total = total + x

    final = total * (1.0 / (NUM_LAYERS + 1))
    return final[:NUM_USERS], final[NUM_USERS:]
